# baseline (speedup 1.0000x reference)
import jax
import jax.numpy as jnp
from jax import lax
from jax.experimental import pallas as pl
from jax.experimental.pallas import tpu as pltpu

BATCH = 4
SEQ = 8192
DIM = 1024


def _pool_body(mask_ref, hs_ref, out_ref, col_v, row_v, sem_rows, sem_m, sem_f):
    mcp = pltpu.make_async_copy(mask_ref.at[:, pl.ds(SEQ - 128, 128)], col_v, sem_m)
    mcp.start()

    spec = pltpu.make_async_copy(hs_ref.at[:, SEQ - 1, :], out_ref, sem_rows)
    spec.start()
    mcp.wait()

    tail = col_v[:, pl.ds(127, 1)]
    all_ok = jnp.logical_and(jnp.min(tail) == 1, jnp.max(tail) == 1)

    spec.wait()

    @pl.when(jnp.logical_not(all_ok))
    def _():
        rcp = pltpu.make_async_copy(mask_ref, row_v, sem_m)
        rcp.start()
        rcp.wait()
        iota_s = lax.broadcasted_iota(jnp.int32, (1, SEQ), 1)
        for b in range(BATCH):
            row = row_v[pl.ds(b, 1), :]
            last = jnp.max(jnp.where(row == 1, iota_s, -1))
            last = jnp.maximum(last, 0)
            fcp = pltpu.make_async_copy(
                hs_ref.at[b].at[pl.ds(last, 1), :],
                out_ref.at[pl.ds(b, 1), :],
                sem_f,
            )
            fcp.start()
            fcp.wait()


def _pool(mask, hidden_states):
    return pl.pallas_call(
        _pool_body,
        out_shape=jax.ShapeDtypeStruct((BATCH, DIM), jnp.float32),
        in_specs=[
            pl.BlockSpec(memory_space=pl.ANY),
            pl.BlockSpec(memory_space=pl.ANY),
        ],
        out_specs=pl.BlockSpec(memory_space=pl.ANY),
        scratch_shapes=[
            pltpu.VMEM((BATCH, 128), jnp.int32),
            pltpu.VMEM((BATCH, SEQ), jnp.int32),
            pltpu.SemaphoreType.DMA,
            pltpu.SemaphoreType.DMA,
            pltpu.SemaphoreType.DMA,
        ],
    )(mask, hidden_states)


def kernel(hidden_states, attention_mask):
    mask = attention_mask.astype(jnp.int32)
    return _pool(mask, hidden_states)


# speedup vs baseline: 1.0508x; 1.0508x over previous
import jax
import jax.numpy as jnp
from jax import lax
from jax.experimental import pallas as pl
from jax.experimental.pallas import tpu as pltpu

BATCH = 4
SEQ = 8192
DIM = 1024


def _pool_body(mask_ref, hs_ref, out_ref, col_v, row_v, sem_rows, sem_m, sem_f):
    mcp = pltpu.make_async_copy(mask_ref.at[:, pl.ds(SEQ - 128, 128)], col_v, sem_m)
    mcp.start()

    spec = pltpu.make_async_copy(hs_ref.at[:, SEQ - 1, :], out_ref, sem_rows)
    spec.start()
    mcp.wait()

    tail = col_v[:, pl.ds(127, 1)]
    all_ok = jnp.logical_and(jnp.min(tail) == 1, jnp.max(tail) == 1)

    spec.wait()

    @pl.when(jnp.logical_not(all_ok))
    def _():
        rcp = pltpu.make_async_copy(mask_ref, row_v, sem_m)
        rcp.start()
        rcp.wait()
        iota_s = lax.broadcasted_iota(jnp.int32, (1, SEQ), 1)
        for b in range(BATCH):
            row = row_v[pl.ds(b, 1), :]
            last = jnp.max(jnp.where(row == 1, iota_s, -1))
            last = jnp.where(last < 0, SEQ - 1, last)
            fcp = pltpu.make_async_copy(
                hs_ref.at[b].at[pl.ds(last, 1), :],
                out_ref.at[pl.ds(b, 1), :],
                sem_f,
            )
            fcp.start()
            fcp.wait()


def _pool(mask, hidden_states):
    return pl.pallas_call(
        _pool_body,
        out_shape=jax.ShapeDtypeStruct((BATCH, DIM), jnp.float32),
        in_specs=[
            pl.BlockSpec(memory_space=pl.ANY),
            pl.BlockSpec(memory_space=pl.ANY),
        ],
        out_specs=pl.BlockSpec((BATCH, DIM), lambda: (0, 0)),
        scratch_shapes=[
            pltpu.VMEM((BATCH, 128), jnp.int32),
            pltpu.VMEM((BATCH, SEQ), jnp.int32),
            pltpu.SemaphoreType.DMA,
            pltpu.SemaphoreType.DMA,
            pltpu.SemaphoreType.DMA,
        ],
    )(mask, hidden_states)


def kernel(hidden_states, attention_mask):
    mask = attention_mask.astype(jnp.int32)
    return _pool(mask, hidden_states)
